# Initial kernel scaffold; baseline (speedup 1.0000x reference)
import jax
import jax.numpy as jnp
from jax.experimental import pallas as pl


def kernel(inputs, step_ids, binary_labels):
    raise NotImplementedError("write your pallas kernel here")


# speedup vs baseline: 9.2131x; 9.2131x over previous
import jax
import jax.numpy as jnp
from jax import lax
from jax.experimental import pallas as pl
from jax.experimental.pallas import tpu as pltpu

B, L, D = 16, 2048, 1024
NS = 8
CHUNK = 1024
NJ = L // CHUNK


def _sums_body(ids_ref, x_ref, o_ref):
    j = pl.program_id(1)
    x = jnp.abs(x_ref[0])
    ids = ids_ref[0]
    iota = lax.broadcasted_iota(jnp.int32, (NS, CHUNK), 0)
    oh = (iota == ids).astype(jnp.float32)
    acc = jnp.dot(oh, x, preferred_element_type=jnp.float32)

    @pl.when(j == 0)
    def _():
        o_ref[0] = acc

    @pl.when(j != 0)
    def _():
        o_ref[0] = o_ref[0] + acc


def _segment_sums(inputs, step_ids):
    ids3 = step_ids.reshape(B * NJ, 1, CHUNK)
    return pl.pallas_call(
        _sums_body,
        grid=(B, NJ),
        in_specs=[
            pl.BlockSpec((1, 1, CHUNK), lambda b, j: (b * NJ + j, 0, 0)),
            pl.BlockSpec((1, CHUNK, D), lambda b, j: (b, j, 0)),
        ],
        out_specs=pl.BlockSpec((1, NS, D), lambda b, j: (b, 0, 0)),
        out_shape=jax.ShapeDtypeStruct((B, NS, D), jnp.float32),
    )(ids3, inputs)


def _loss_body(ids_ref, sums_ref, lab_ref, o_ref):
    ids = ids_ref[...]
    pos = lax.broadcasted_iota(jnp.int32, (B, L), 1)
    cnt_cols, first_cols = [], []
    for s in range(1, NS):
        eq = ids == s
        cnt_cols.append(jnp.sum(eq.astype(jnp.float32), axis=1, keepdims=True))
        first_cols.append(jnp.min(jnp.where(eq, pos, L), axis=1, keepdims=True))
    cnt = jnp.concatenate(cnt_cols, axis=1)
    first = jnp.concatenate(first_cols, axis=1)

    steps_row = lax.broadcasted_iota(jnp.int32, (1, NS - 1), 1) + 1
    key = first * NS + steps_row
    rank = jnp.sum((key[:, None, :] < key[:, :, None]).astype(jnp.int32),
                   axis=2)

    sums = sums_ref[...]
    means = sums[:, 1:, :] / jnp.maximum(cnt, 1.0)[:, :, None]

    Hs, vals = [], []
    for r in range(NS - 1):
        sel = (rank == r).astype(jnp.float32)
        Hs.append(jnp.sum(sel[:, :, None] * means, axis=1))
        vals.append(jnp.sum(sel * steps_row.astype(jnp.float32),
                            axis=1, keepdims=True))

    K = jnp.sum((cnt > 0).astype(jnp.int32), axis=1, keepdims=True)
    termA = jnp.zeros((B, 1), jnp.float32)
    termB = jnp.zeros((B, 1), jnp.float32)
    dcnt = jnp.zeros((B, 1), jnp.float32)
    for i in range(NS - 2):
        d = jnp.maximum(Hs[i] - Hs[i + 1], 0.0)
        E = jnp.mean(d * d, axis=1, keepdims=True)
        valid = ((i + 1) < K)
        desc = (vals[i] > vals[i + 1]) & valid
        descf = desc.astype(jnp.float32)
        dcnt = dcnt + descf
        termA = termA + E * valid.astype(jnp.float32)
        termB = termB + jnp.maximum(1.0 - E, 0.0) * descf
    termA = termA / jnp.maximum(K.astype(jnp.float32) - 1.0, 1.0)
    termB = termB / jnp.maximum(dcnt, 1.0)

    lab = lab_ref[...]
    hasA = (lab == 1) & (K >= 2)
    hasB = (lab == 0) & (dcnt > 0)
    totalb = (jnp.where(hasA, termA, 0.0) + jnp.where(hasB, termB, 0.0))
    numb = hasA.astype(jnp.float32) + hasB.astype(jnp.float32)
    total = jnp.sum(totalb)
    num = jnp.sum(numb)
    res = total / (num + 1e-9)
    o_ref[...] = jnp.full((8, 128), res, jnp.float32)


def _loss_epilogue(step_ids, sums, binary_labels):
    out = pl.pallas_call(
        _loss_body,
        out_shape=jax.ShapeDtypeStruct((8, 128), jnp.float32),
    )(step_ids, sums, binary_labels.reshape(B, 1))
    return out[0, 0]


def kernel(inputs, step_ids, binary_labels):
    sums = _segment_sums(inputs, step_ids)
    return _loss_epilogue(step_ids, sums, binary_labels)
